# baseline (speedup 1.0000x reference)
import jax
import jax.numpy as jnp
from jax.experimental import pallas as pl
from jax.experimental.pallas import tpu as pltpu


def _mm_kernel(idx_ref, x_ref, *rest):
    del idx_ref
    n = (len(rest) - 1) // 2
    w_refs = rest[:n]
    b_refs = rest[n:2 * n]
    o_ref = rest[-1]
    xa = x_ref[0]
    for s in range(n):
        acc = jax.lax.dot_general(
            xa, w_refs[s][0],
            dimension_numbers=(((1,), (1,)), ((), ())),
            preferred_element_type=jnp.float32,
        )
        o_ref[0, s] = acc + b_refs[s][0]


def kernel(x, indices, weight, bias):
    G, S = indices.shape
    E, OUT_D, IN_D = weight.shape
    T = x.shape[-2]

    xr = x.reshape(G, T, IN_D)
    br = bias.reshape(E, 1, OUT_D)

    def _wmap(s):
        return lambda i, ind: (ind[i, s], 0, 0)

    grid_spec = pltpu.PrefetchScalarGridSpec(
        num_scalar_prefetch=1,
        grid=(G,),
        in_specs=[
            pl.BlockSpec((1, T, IN_D), lambda i, ind: (i, 0, 0)),
        ] + [
            pl.BlockSpec((1, OUT_D, IN_D), _wmap(s)) for s in range(S)
        ] + [
            pl.BlockSpec((1, 1, OUT_D), _wmap(s)) for s in range(S)
        ],
        out_specs=pl.BlockSpec((1, S, T, OUT_D),
                               lambda i, ind: (i, 0, 0, 0)),
    )

    out = pl.pallas_call(
        _mm_kernel,
        grid_spec=grid_spec,
        out_shape=jax.ShapeDtypeStruct((G, S, T, OUT_D), jnp.float32),
        compiler_params=pltpu.CompilerParams(
            dimension_semantics=("parallel",),
        ),
    )(indices, xr, *([weight] * S), *([br] * S))

    return out.reshape(1, G, S, T, OUT_D)


# speedup vs baseline: 1.0393x; 1.0393x over previous
import jax
import jax.numpy as jnp
from jax.experimental import pallas as pl
from jax.experimental.pallas import tpu as pltpu

_XCHUNKS = 4


def _mm_kernel(idx_ref, fu_ref, eord_ref, mask_ref,
               x_hbm, w_hbm, b_ref, o_ref,
               xscr, wscr, xsem, wsemA, wsemB):
    G, T, IN_D = xscr.shape
    E, OUT_D = wscr.shape[0], wscr.shape[1]
    HALF = OUT_D // 2
    S = idx_ref.shape[1]
    rows = G // _XCHUNKS
    i = pl.program_id(0)

    def _xcopy(c):
        return pltpu.make_async_copy(
            x_hbm.at[pl.ds(c * rows, rows)],
            xscr.at[pl.ds(c * rows, rows)],
            xsem.at[c],
        )

    def _wcopyA(e):
        return pltpu.make_async_copy(
            w_hbm.at[e, pl.ds(0, HALF)],
            wscr.at[e, pl.ds(0, HALF)],
            wsemA.at[e],
        )

    def _wcopyB(e):
        return pltpu.make_async_copy(
            w_hbm.at[e, pl.ds(HALF, HALF)],
            wscr.at[e, pl.ds(HALF, HALF)],
            wsemB.at[e],
        )

    @pl.when(i == 0)
    def _issue():
        _xcopy(0).start()
        for k in range(E):
            e = eord_ref[k]

            @pl.when(mask_ref[k] == 1)
            def _start_w():
                _wcopyA(e).start()
                _wcopyB(e).start()

            if k + 1 < _XCHUNKS:
                _xcopy(k + 1).start()

    @pl.when(i % rows == 0)
    def _wait_x():
        _xcopy(i // rows).wait()

    for s in range(S):
        e_s = idx_ref[i, s]

        @pl.when(fu_ref[i, s] == 1)
        def _wait_w():
            _wcopyA(e_s).wait()
            _wcopyB(e_s).wait()

        acc = jax.lax.dot_general(
            xscr[i], wscr[e_s],
            dimension_numbers=(((1,), (1,)), ((), ())),
            preferred_element_type=jnp.float32,
        )
        o_ref[0, s] = acc + b_ref[e_s]


def kernel(x, indices, weight, bias):
    G, S = indices.shape
    E, OUT_D, IN_D = weight.shape
    T = x.shape[-2]
    P = G * S

    xr = x.reshape(G, T, IN_D)

    flat = indices.reshape(P)
    eq = flat[:, None] == flat[None, :]
    first = jnp.argmax(eq, axis=1)
    fu = (first == jnp.arange(P)).astype(jnp.int32).reshape(G, S)
    onehot = flat[None, :] == jnp.arange(E)[:, None]
    firstpos = jnp.where(onehot, jnp.arange(P)[None, :], P).min(axis=1)
    eord = jnp.argsort(firstpos).astype(jnp.int32)
    mask = (jnp.sort(firstpos) < P).astype(jnp.int32)

    grid_spec = pltpu.PrefetchScalarGridSpec(
        num_scalar_prefetch=4,
        grid=(G,),
        in_specs=[
            pl.BlockSpec(memory_space=pl.ANY),
            pl.BlockSpec(memory_space=pl.ANY),
            pl.BlockSpec((E, OUT_D), lambda i, *_: (0, 0)),
        ],
        out_specs=pl.BlockSpec((1, S, T, OUT_D),
                               lambda i, *_: (i, 0, 0, 0)),
        scratch_shapes=[
            pltpu.VMEM((G, T, IN_D), jnp.float32),
            pltpu.VMEM((E, OUT_D, IN_D), jnp.float32),
            pltpu.SemaphoreType.DMA((_XCHUNKS,)),
            pltpu.SemaphoreType.DMA((E,)),
            pltpu.SemaphoreType.DMA((E,)),
        ],
    )

    out = pl.pallas_call(
        _mm_kernel,
        grid_spec=grid_spec,
        out_shape=jax.ShapeDtypeStruct((G, S, T, OUT_D), jnp.float32),
    )(indices, fu, eord, mask, xr, weight, bias)

    return out.reshape(1, G, S, T, OUT_D)


# speedup vs baseline: 1.1208x; 1.0784x over previous
import jax
import jax.numpy as jnp
from jax.experimental import pallas as pl
from jax.experimental.pallas import tpu as pltpu

_XCHUNKS = 4


def _mm_kernel(idx_ref, fu_ref, eord_ref, mask_ref,
               x_hbm, w_hbm, b_ref, o_ref,
               xscr, wscr, xsem, wsemA):
    G, T, IN_D = xscr.shape
    E, OUT_D = wscr.shape[0], wscr.shape[1]
    HALF = OUT_D // 2
    S = idx_ref.shape[1]
    rows = G // _XCHUNKS
    i = pl.program_id(0)

    def _xcopy(c):
        return pltpu.make_async_copy(
            x_hbm.at[pl.ds(c * rows, rows)],
            xscr.at[pl.ds(c * rows, rows)],
            xsem.at[c],
        )

    def _wcopy(e):
        return pltpu.make_async_copy(
            w_hbm.at[e], wscr.at[e], wsemA.at[e])

    @pl.when(i == 0)
    def _issue():
        for c in range(_XCHUNKS):
            _xcopy(c).start()
        for k in range(E):
            e = eord_ref[k]

            @pl.when(mask_ref[k] == 1)
            def _start_w():
                _wcopy(e).start()

    @pl.when(i % rows == 0)
    def _wait_x():
        _xcopy(i // rows).wait()

    for s in range(S):
        e_s = idx_ref[i, s]

        @pl.when(fu_ref[i, s] == 1)
        def _wait_w():
            _wcopy(e_s).wait()

        acc = jax.lax.dot_general(
            xscr[i], wscr[e_s],
            dimension_numbers=(((1,), (1,)), ((), ())),
            preferred_element_type=jnp.float32,
        )
        o_ref[0, s] = acc + b_ref[e_s]


def kernel(x, indices, weight, bias):
    G, S = indices.shape
    E, OUT_D, IN_D = weight.shape
    T = x.shape[-2]
    P = G * S

    xr = x.reshape(G, T, IN_D)

    flat = indices.reshape(P)
    eq = flat[:, None] == flat[None, :]
    first = jnp.argmax(eq, axis=1)
    fu = (first == jnp.arange(P)).astype(jnp.int32).reshape(G, S)
    onehot = flat[None, :] == jnp.arange(E)[:, None]
    firstpos = jnp.where(onehot, jnp.arange(P)[None, :], P).min(axis=1)
    eord = jnp.argsort(firstpos).astype(jnp.int32)
    mask = (jnp.sort(firstpos) < P).astype(jnp.int32)

    grid_spec = pltpu.PrefetchScalarGridSpec(
        num_scalar_prefetch=4,
        grid=(G,),
        in_specs=[
            pl.BlockSpec(memory_space=pl.ANY),
            pl.BlockSpec(memory_space=pl.ANY),
            pl.BlockSpec((E, OUT_D), lambda i, *_: (0, 0)),
        ],
        out_specs=pl.BlockSpec((1, S, T, OUT_D),
                               lambda i, *_: (i, 0, 0, 0)),
        scratch_shapes=[
            pltpu.VMEM((G, T, IN_D), jnp.float32),
            pltpu.VMEM((E, OUT_D, IN_D), jnp.float32),
            pltpu.SemaphoreType.DMA((_XCHUNKS,)),
            pltpu.SemaphoreType.DMA((E,)),
        ],
    )

    out = pl.pallas_call(
        _mm_kernel,
        grid_spec=grid_spec,
        out_shape=jax.ShapeDtypeStruct((G, S, T, OUT_D), jnp.float32),
    )(indices, fu, eord, mask, xr, weight, bias)

    return out.reshape(1, G, S, T, OUT_D)
